# baseline (speedup 1.0000x reference)
import functools

import jax
import jax.numpy as jnp
from jax import lax
from jax.experimental import pallas as pl
from jax.experimental.pallas import tpu as pltpu
from jax.experimental.pallas import tpu_sc as plsc

S, B, D, K, V = 50, 1024, 128, 10, 100000
SB = S * B
N = SB * K
NC, NS = 2, 16
NW = NC * NS
PAIRS_PER_W = SB // NW
IDS_PER_W = PAIRS_PER_W * K
NBUF = 4
CP = 16
CK = CP * K
NCHUNK = PAIRS_PER_W // CP
STREAMS = ((0, 128), (128, 32))
LANES = 16
DSUB = D // LANES
BSUB = D // (2 * LANES)
DW = D // 2


def _sc_neg_dots(ids_hbm, ctx_hbm, table_hbm, out_hbm, idx_v, dots_v,
                 rows_v, ctx_v, sg0, sg1, sg2, sg3, sc0, sc1, sc2, sc3):
    wid = lax.axis_index("s") * NC + lax.axis_index("c")
    pair0 = wid * PAIRS_PER_W
    id0 = pair0 * K
    lane_iota = lax.iota(jnp.int32, LANES)
    last_lane = lane_iota == (LANES - 1)
    sgs = (sg0, sg1, sg2, sg3)
    scs = (sc0, sc1, sc2, sc3)

    pltpu.sync_copy(ids_hbm.at[pl.ds(id0, IDS_PER_W)], idx_v)

    def issue(g, buf):
        for (off, n) in STREAMS:
            pltpu.async_copy(
                table_hbm.at[idx_v.at[pl.ds(g * CK + off, n)]],
                rows_v.at[pl.ds(buf * CK + off, n)], sgs[buf])
        pltpu.async_copy(ctx_hbm.at[pl.ds(pair0 + g * CP, CP)],
                         ctx_v.at[pl.ds(buf * CP, CP)], scs[buf])

    def drain(buf):
        for (off, n) in STREAMS:
            pltpu.make_async_copy(table_hbm.at[pl.ds(0, n)],
                                  rows_v.at[pl.ds(buf * CK + off, n)],
                                  sgs[buf]).wait()
        pltpu.make_async_copy(ctx_hbm.at[pl.ds(0, CP)],
                              ctx_v.at[pl.ds(buf * CP, CP)], scs[buf]).wait()

    for b in range(NBUF):
        issue(b, b)

    def body(g, carry):
        par = lax.rem(g, NBUF)

        for b in range(NBUF):
            @pl.when(par == b)
            def _drain(b=b):
                drain(b)

        rbase = par * CK
        cbase = par * CP

        @plsc.parallel_loop(0, CP, 1, unroll=3)
        def pair_body(p):
            c = [ctx_v[cbase + p, pl.ds(j * LANES, LANES)]
                 for j in range(DSUB)]
            for k in range(K):
                t = p * K + k
                pr = [rows_v[rbase + t, pl.ds(j * LANES, LANES)] * c[j]
                      for j in range(DSUB)]
                acc = (((pr[0] + pr[1]) + (pr[2] + pr[3]))
                       + ((pr[4] + pr[5]) + (pr[6] + pr[7])))
                plsc.store_compressed(dots_v.at[pl.ds(g * CK + t, LANES)],
                                      plsc.cumsum(acc), mask=last_lane)

        @pl.when(g < NCHUNK - NBUF)
        def _refill():
            for b in range(NBUF):
                @pl.when(par == b)
                def _r(b=b):
                    issue(g + NBUF, b)

        return carry

    lax.fori_loop(0, NCHUNK, body, 0)
    pltpu.sync_copy(dots_v.at[pl.ds(0, IDS_PER_W)],
                    out_hbm.at[pl.ds(id0, IDS_PER_W)])


@functools.cache
def _make_neg_dots():
    return functools.partial(
        pl.kernel,
        mesh=plsc.VectorSubcoreMesh(core_axis_name="c", subcore_axis_name="s"),
        out_type=jax.ShapeDtypeStruct((N,), jnp.float32),
        compiler_params=pltpu.CompilerParams(needs_layout_passes=False),
        scratch_types=[
            pltpu.VMEM((IDS_PER_W,), jnp.int32),
            pltpu.VMEM((IDS_PER_W + LANES,), jnp.float32),
            pltpu.VMEM((NBUF * CK, D), jnp.float32),
            pltpu.VMEM((NBUF * CP, D), jnp.float32),
            pltpu.SemaphoreType.DMA,
            pltpu.SemaphoreType.DMA,
            pltpu.SemaphoreType.DMA,
            pltpu.SemaphoreType.DMA,
            pltpu.SemaphoreType.DMA,
            pltpu.SemaphoreType.DMA,
            pltpu.SemaphoreType.DMA,
            pltpu.SemaphoreType.DMA,
        ],
    )(_sc_neg_dots)


def _pos_body(pos_ref, ctx_ref, out_ref):
    i = pl.program_id(0)

    @pl.when(i == 0)
    def _init():
        out_ref[0, 0] = 0.0

    pos_logits = jnp.sum(pos_ref[0] * ctx_ref[0], axis=-1)
    out_ref[0, 0] += jnp.sum(jax.nn.log_sigmoid(pos_logits))


def _neg_body(dots_ref, out_ref):
    out_ref[0, 0] = jnp.sum(jax.nn.log_sigmoid(-dots_ref[...]))


def kernel(positive_sample, context_tensor, emb_table, negative_sample_ids):
    ids32 = negative_sample_ids.astype(jnp.int32).reshape(N)
    ctx2d = context_tensor.reshape(SB, D)

    pos_part = pl.pallas_call(
        _pos_body,
        grid=(S,),
        in_specs=[
            pl.BlockSpec((1, B, D), lambda i: (i, 0, 0)),
            pl.BlockSpec((1, B, D), lambda i: (i, 0, 0)),
        ],
        out_specs=pl.BlockSpec((1, 1), lambda i: (0, 0),
                               memory_space=pltpu.SMEM),
        out_shape=jax.ShapeDtypeStruct((1, 1), jnp.float32),
    )(positive_sample, context_tensor)

    dots = _make_neg_dots()(ids32, ctx2d, emb_table)

    neg_part = pl.pallas_call(
        _neg_body,
        out_specs=pl.BlockSpec(memory_space=pltpu.SMEM),
        out_shape=jax.ShapeDtypeStruct((1, 1), jnp.float32),
    )(dots)

    return -(pos_part[0, 0] + neg_part[0, 0])


# speedup vs baseline: 1.9857x; 1.9857x over previous
import functools

import jax
import jax.numpy as jnp
from jax import lax
from jax.experimental import pallas as pl
from jax.experimental.pallas import tpu as pltpu
from jax.experimental.pallas import tpu_sc as plsc

S, B, D, K, V = 50, 1024, 128, 10, 100000
SB = S * B
N = SB * K
NC, NS = 2, 16
NW = NC * NS
PAIRS_PER_W = SB // NW
IDS_PER_W = PAIRS_PER_W * K
NBUF = 4
CP = 16
CK = CP * K
NCHUNK = PAIRS_PER_W // CP
STREAMS = ((0, 128), (128, 32))
LANES = 16
DSUB = D // LANES
BSUB = D // (2 * LANES)
DW = D // 2


def _sc_neg_dots(ids_hbm, ctx_hbm, table_hbm, out_hbm, idx_v, dots_v,
                 rows_v, ctx_v, sg0, sg1, sg2, sg3, sc0, sc1, sc2, sc3):
    wid = lax.axis_index("s") * NC + lax.axis_index("c")
    pair0 = wid * PAIRS_PER_W
    id0 = pair0 * K
    lane_iota = lax.iota(jnp.int32, LANES)
    last_lane = lane_iota == (LANES - 1)
    sgs = (sg0, sg1, sg2, sg3)
    scs = (sc0, sc1, sc2, sc3)

    pltpu.sync_copy(ids_hbm.at[pl.ds(id0, IDS_PER_W)], idx_v)

    def issue(g, buf):
        for (off, n) in STREAMS:
            pltpu.async_copy(
                table_hbm.at[idx_v.at[pl.ds(g * CK + off, n)]],
                rows_v.at[pl.ds(buf * CK + off, n)], sgs[buf])
        pltpu.async_copy(ctx_hbm.at[pl.ds(pair0 + g * CP, CP)],
                         ctx_v.at[pl.ds(buf * CP, CP)], scs[buf])

    def drain(buf):
        for (off, n) in STREAMS:
            pltpu.make_async_copy(table_hbm.at[pl.ds(0, n)],
                                  rows_v.at[pl.ds(buf * CK + off, n)],
                                  sgs[buf]).wait()
        pltpu.make_async_copy(ctx_hbm.at[pl.ds(0, CP)],
                              ctx_v.at[pl.ds(buf * CP, CP)], scs[buf]).wait()

    for b in range(NBUF):
        issue(b, b)

    def body(g, carry):
        par = lax.rem(g, NBUF)

        for b in range(NBUF):
            @pl.when(par == b)
            def _drain(b=b):
                drain(b)

        rbase = par * CK
        cbase = par * CP

        @plsc.parallel_loop(0, CP, 1, unroll=2)
        def pair_body(p):
            c = [ctx_v[cbase + p, pl.ds(j * LANES, LANES)]
                 for j in range(DSUB)]
            for k in range(K):
                t = p * K + k
                pr = [rows_v[rbase + t, pl.ds(j * LANES, LANES)] * c[j]
                      for j in range(DSUB)]
                acc = (((pr[0] + pr[1]) + (pr[2] + pr[3]))
                       + ((pr[4] + pr[5]) + (pr[6] + pr[7])))
                plsc.store_compressed(dots_v.at[pl.ds(g * CK + t, LANES)],
                                      plsc.cumsum(acc), mask=last_lane)

        @pl.when(g < NCHUNK - NBUF)
        def _refill():
            for b in range(NBUF):
                @pl.when(par == b)
                def _r(b=b):
                    issue(g + NBUF, b)

        return carry

    lax.fori_loop(0, NCHUNK, body, 0)
    pltpu.sync_copy(dots_v.at[pl.ds(0, IDS_PER_W)],
                    out_hbm.at[pl.ds(id0, IDS_PER_W)])


@functools.cache
def _make_neg_dots():
    return functools.partial(
        pl.kernel,
        mesh=plsc.VectorSubcoreMesh(core_axis_name="c", subcore_axis_name="s"),
        out_type=jax.ShapeDtypeStruct((N,), jnp.float32),
        compiler_params=pltpu.CompilerParams(needs_layout_passes=False),
        scratch_types=[
            pltpu.VMEM((IDS_PER_W,), jnp.int32),
            pltpu.VMEM((IDS_PER_W + LANES,), jnp.float32),
            pltpu.VMEM((NBUF * CK, D), jnp.float32),
            pltpu.VMEM((NBUF * CP, D), jnp.float32),
            pltpu.SemaphoreType.DMA,
            pltpu.SemaphoreType.DMA,
            pltpu.SemaphoreType.DMA,
            pltpu.SemaphoreType.DMA,
            pltpu.SemaphoreType.DMA,
            pltpu.SemaphoreType.DMA,
            pltpu.SemaphoreType.DMA,
            pltpu.SemaphoreType.DMA,
        ],
    )(_sc_neg_dots)


def _pos_body(pos_ref, ctx_ref, out_ref):
    i = pl.program_id(0)

    @pl.when(i == 0)
    def _init():
        out_ref[0, 0] = 0.0

    pos_logits = jnp.sum(pos_ref[0] * ctx_ref[0], axis=-1)
    out_ref[0, 0] += jnp.sum(jax.nn.log_sigmoid(pos_logits))


def _neg_body(dots_ref, out_ref):
    out_ref[0, 0] = jnp.sum(jax.nn.log_sigmoid(-dots_ref[...]))


def kernel(positive_sample, context_tensor, emb_table, negative_sample_ids):
    ids32 = negative_sample_ids.astype(jnp.int32).reshape(N)
    ctx2d = context_tensor.reshape(SB, D)

    pos_part = pl.pallas_call(
        _pos_body,
        grid=(S,),
        in_specs=[
            pl.BlockSpec((1, B, D), lambda i: (i, 0, 0)),
            pl.BlockSpec((1, B, D), lambda i: (i, 0, 0)),
        ],
        out_specs=pl.BlockSpec((1, 1), lambda i: (0, 0),
                               memory_space=pltpu.SMEM),
        out_shape=jax.ShapeDtypeStruct((1, 1), jnp.float32),
    )(positive_sample, context_tensor)

    dots = _make_neg_dots()(ids32, ctx2d, emb_table)

    neg_part = pl.pallas_call(
        _neg_body,
        out_specs=pl.BlockSpec(memory_space=pltpu.SMEM),
        out_shape=jax.ShapeDtypeStruct((1, 1), jnp.float32),
    )(dots)

    return -(pos_part[0, 0] + neg_part[0, 0])
